# baseline (speedup 1.0000x reference)
import jax
import jax.numpy as jnp
from jax import lax
from jax.experimental import pallas as pl
from jax.experimental.pallas import tpu as pltpu
from jax.experimental.pallas import tpu_sc as plsc

T = 2048
D = 2048
E = 16
K = 8
F = 768

_RT = 256
_TMG = 256
_TMD = 1024
_C = T
_P = E * _C
_JT = _C // _TMG
_DW = D // 2
_FW = F // 2
_TRASH = _C + 8


def _router_body(x_ref, wr_ref, selt_ref, combt_ref):
    x = x_ref[...].astype(jnp.bfloat16)
    w = wr_ref[...].astype(jnp.bfloat16)
    logits = jax.lax.dot_general(
        x, w, (((1,), (0,)), ((), ())), preferred_element_type=jnp.float32)
    m = jnp.max(logits, axis=-1, keepdims=True)
    ex = jnp.exp(logits - m)
    probs = ex / jnp.sum(ex, axis=-1, keepdims=True)
    p = probs
    sel = jnp.zeros(p.shape, dtype=jnp.bool_)
    idx = jax.lax.broadcasted_iota(jnp.int32, p.shape, 1)
    for _ in range(K):
        mx = jnp.max(p, axis=-1, keepdims=True)
        ismx = p == mx
        first_idx = jnp.min(jnp.where(ismx, idx, E), axis=-1, keepdims=True)
        pick = idx == first_idx
        sel = sel | pick
        p = jnp.where(pick, -jnp.inf, p)
    w8 = jnp.where(sel, probs, 0.0)
    comb = w8 / jnp.sum(w8, axis=-1, keepdims=True)
    selt_ref[...] = jnp.transpose(sel.astype(jnp.int32))
    combt_ref[...] = jnp.transpose(comb)


def _dispatch_body(selt, combt, xb32, st_hbm, ws_hbm, cnts_hbm, zd_hbm,
                   xs_hbm, selv, combv, toksv, wv, zdv, cntv, idxv,
                   rb0, rb1, rb2, g0, g1, g2, s0, s1, s2):
    cid = lax.axis_index("c")
    sid = lax.axis_index("s")

    @pl.when(sid < 8)
    def _meta():
        e = cid * 8 + sid
        pltpu.sync_copy(selt.at[e], selv)
        pltpu.sync_copy(combt.at[e], combv)

        def scan_sel(j, cnt):
            s16 = selv[pl.ds(j * 16, 16)]
            mvec = s16 != 0
            mi = mvec.astype(jnp.int32)
            excl = plsc.cumsum(mi) - mi
            dest = jnp.where(mvec, cnt + excl, _TRASH)
            toks = lax.broadcasted_iota(jnp.int32, (16,), 0) + j * 16
            plsc.store_scatter(toksv, [dest], toks)
            plsc.store_scatter(wv, [dest], combv[pl.ds(j * 16, 16)])
            return cnt + jnp.sum(mi)
        cnt = lax.fori_loop(0, _C // 16, scan_sel, jnp.int32(0))

        def scan_unsel(j, c2):
            s16 = selv[pl.ds(j * 16, 16)]
            mvec = s16 == 0
            mi = mvec.astype(jnp.int32)
            excl = plsc.cumsum(mi) - mi
            dest = jnp.where(mvec, c2 + excl, _TRASH)
            toks = lax.broadcasted_iota(jnp.int32, (16,), 0) + j * 16
            plsc.store_scatter(toksv, [dest], toks)
            plsc.store_scatter(wv, [dest], jnp.zeros((16,), jnp.float32))
            return c2 + jnp.sum(mi)
        lax.fori_loop(0, _C // 16, scan_unsel, cnt)

        def zmap(j, _):
            t16 = toksv[pl.ds(j * 16, 16)]
            zdv[pl.ds(j * 16, 16)] = t16 * E + e
            return 0
        lax.fori_loop(0, _C // 16, zmap, 0)

        cntv[...] = jnp.full((16,), cnt, jnp.int32)
        pltpu.sync_copy(cntv, cnts_hbm.at[e])
        pltpu.sync_copy(toksv.at[pl.ds(0, _C)], st_hbm.at[pl.ds(e * _C, _C)])
        pltpu.sync_copy(wv.at[pl.ds(0, _C)], ws_hbm.at[pl.ds(e * _C, _C)])
        pltpu.sync_copy(zdv, zd_hbm.at[pl.ds(e * _C, _C)])

    plsc.subcore_barrier()

    e = cid * 8 + sid // 2
    half = sid % 2
    base = e * _C + half * 1024
    pltpu.sync_copy(st_hbm.at[pl.ds(base, 1024)], idxv)
    pltpu.sync_copy(cnts_hbm.at[e], cntv)
    cnt = jnp.max(cntv[...])
    v = jnp.clip(cnt - half * 1024, 0, 1024)
    nch = (v + 15) // 16

    rbs = (rb0, rb1, rb2)
    gsems = (g0, g1, g2)
    ssems = (s0, s1, s2)

    def issue_gather(k, b):
        i16 = idxv[pl.ds(k * 16, 16)]
        pltpu.async_copy(xb32.at[i16], rbs[b], gsems[b])

    for i in range(3):
        @pl.when(i < nch)
        def _prime(i=i):
            issue_gather(i, i)

    def chunk(k, _):
        for b in range(3):
            @pl.when(k % 3 == b)
            def _do(b=b):
                pltpu.make_async_copy(
                    xb32.at[pl.ds(0, 16)], rbs[b], gsems[b]).wait()
                pltpu.async_copy(
                    rbs[b], xs_hbm.at[pl.ds(base + k * 16, 16)],
                    ssems[b]).wait()
                @pl.when(k + 3 < nch)
                def _next():
                    issue_gather(k + 3, b)
        return 0
    lax.fori_loop(0, nch, chunk, 0)


def _gateup_body(cnt_ref, xs_ref, wg_ref, wu_ref, w_ref, h_ref):
    e = pl.program_id(0)
    j = pl.program_id(1)
    cnt = cnt_ref[e]

    @pl.when(j * _TMG < cnt)
    def _():
        x = xs_ref[...]
        g = jax.lax.dot_general(
            x, wg_ref[0], (((1,), (0,)), ((), ())),
            preferred_element_type=jnp.float32)
        u = jax.lax.dot_general(
            x, wu_ref[0], (((1,), (0,)), ((), ())),
            preferred_element_type=jnp.float32)
        h = (g * jax.lax.logistic(g)) * u
        h_ref[...] = (h * jnp.transpose(w_ref[0])).astype(jnp.bfloat16)

    @pl.when(j * _TMG >= cnt)
    def _():
        h_ref[...] = jnp.zeros((_TMG, F), jnp.bfloat16)


def _xs_index(e, j, cnt_ref):
    jcap = jnp.maximum(pl.cdiv(cnt_ref[e], _TMG) - 1, 0)
    return (e * _JT + jnp.minimum(j, jcap), 0)


def _scatter_body(h32, zd_hbm, z_hbm, idxv, rb0, rb1, rb2,
                  g0, g1, g2, s0, s1, s2):
    cid = lax.axis_index("c")
    sid = lax.axis_index("s")
    wid = sid * 2 + cid
    base = wid * 1024
    pltpu.sync_copy(zd_hbm.at[pl.ds(base, 1024)], idxv)

    rbs = (rb0, rb1, rb2)
    gsems = (g0, g1, g2)
    ssems = (s0, s1, s2)
    nch = 64

    def issue_read(k, b):
        pltpu.async_copy(h32.at[pl.ds(base + k * 16, 16)], rbs[b], gsems[b])

    for i in range(3):
        issue_read(i, i)

    def chunk(k, _):
        for b in range(3):
            @pl.when(k % 3 == b)
            def _do(b=b):
                pltpu.make_async_copy(
                    h32.at[pl.ds(0, 16)], rbs[b], gsems[b]).wait()
                i16 = idxv[pl.ds(k * 16, 16)]
                pltpu.async_copy(rbs[b], z_hbm.at[i16], ssems[b]).wait()
                @pl.when(k + 3 < nch)
                def _next():
                    issue_read(k + 3, b)
        return 0
    lax.fori_loop(0, nch, chunk, 0)


def _down_body(zb_ref, wd_ref, out_ref):
    e = pl.program_id(1)
    y = jax.lax.dot_general(
        zb_ref[...], wd_ref[...], (((1,), (0,)), ((), ())),
        preferred_element_type=jnp.float32)

    @pl.when(e == 0)
    def _():
        out_ref[...] = y

    @pl.when(e != 0)
    def _():
        out_ref[...] += y


def kernel(x, W_router, w_gate, w_up, w_down):
    selt, combt = pl.pallas_call(
        _router_body,
        grid=(T // _RT,),
        in_specs=[
            pl.BlockSpec((_RT, D), lambda i: (i, 0)),
            pl.BlockSpec((D, E), lambda i: (0, 0)),
        ],
        out_specs=[
            pl.BlockSpec((E, _RT), lambda i: (0, i)),
            pl.BlockSpec((E, _RT), lambda i: (0, i)),
        ],
        out_shape=[
            jax.ShapeDtypeStruct((E, T), jnp.int32),
            jax.ShapeDtypeStruct((E, T), jnp.float32),
        ],
    )(x, W_router)

    xb = x.astype(jnp.bfloat16)
    xb32 = jax.lax.bitcast_convert_type(xb.reshape(T, _DW, 2), jnp.int32)

    mesh = plsc.VectorSubcoreMesh(core_axis_name="c", subcore_axis_name="s")
    scp = pltpu.CompilerParams(needs_layout_passes=False)
    dispatch = pl.kernel(
        _dispatch_body,
        out_type=[
            jax.ShapeDtypeStruct((_P,), jnp.int32),
            jax.ShapeDtypeStruct((_P,), jnp.float32),
            jax.ShapeDtypeStruct((E, 16), jnp.int32),
            jax.ShapeDtypeStruct((_P,), jnp.int32),
            jax.ShapeDtypeStruct((_P, _DW), jnp.int32),
        ],
        mesh=mesh,
        compiler_params=scp,
        scratch_types=[
            pltpu.VMEM((_C,), jnp.int32),
            pltpu.VMEM((_C,), jnp.float32),
            pltpu.VMEM((_C + 24,), jnp.int32),
            pltpu.VMEM((_C + 24,), jnp.float32),
            pltpu.VMEM((_C,), jnp.int32),
            pltpu.VMEM((16,), jnp.int32),
            pltpu.VMEM((1024,), jnp.int32),
            pltpu.VMEM((16, _DW), jnp.int32),
            pltpu.VMEM((16, _DW), jnp.int32),
            pltpu.VMEM((16, _DW), jnp.int32),
            pltpu.SemaphoreType.DMA,
            pltpu.SemaphoreType.DMA,
            pltpu.SemaphoreType.DMA,
            pltpu.SemaphoreType.DMA,
            pltpu.SemaphoreType.DMA,
            pltpu.SemaphoreType.DMA,
        ],
    )
    sorted_tok, w_sorted, counts2d, zdest, xs32 = dispatch(selt, combt, xb32)
    del sorted_tok

    xs_bf = jax.lax.bitcast_convert_type(xs32, jnp.bfloat16).reshape(_P, D)
    counts = counts2d[:, 0]
    w3d = w_sorted.reshape(_P // _TMG, 1, _TMG)
    wgb = w_gate.astype(jnp.bfloat16)
    wub = w_up.astype(jnp.bfloat16)

    h_bf = pl.pallas_call(
        _gateup_body,
        grid_spec=pltpu.PrefetchScalarGridSpec(
            num_scalar_prefetch=1,
            grid=(E, _JT),
            in_specs=[
                pl.BlockSpec((_TMG, D), _xs_index),
                pl.BlockSpec((1, D, F), lambda e, j, c: (e, 0, 0)),
                pl.BlockSpec((1, D, F), lambda e, j, c: (e, 0, 0)),
                pl.BlockSpec((1, 1, _TMG), lambda e, j, c: (e * _JT + j, 0, 0)),
            ],
            out_specs=pl.BlockSpec((_TMG, F), lambda e, j, c: (e * _JT + j, 0)),
        ),
        out_shape=jax.ShapeDtypeStruct((_P, F), jnp.bfloat16),
    )(counts, xs_bf, wgb, wub, w3d)

    h32 = jax.lax.bitcast_convert_type(h_bf.reshape(_P, _FW, 2), jnp.int32)

    scatter = pl.kernel(
        _scatter_body,
        out_type=jax.ShapeDtypeStruct((_P, _FW), jnp.int32),
        mesh=mesh,
        compiler_params=scp,
        scratch_types=[
            pltpu.VMEM((1024,), jnp.int32),
            pltpu.VMEM((16, _FW), jnp.int32),
            pltpu.VMEM((16, _FW), jnp.int32),
            pltpu.VMEM((16, _FW), jnp.int32),
            pltpu.SemaphoreType.DMA,
            pltpu.SemaphoreType.DMA,
            pltpu.SemaphoreType.DMA,
            pltpu.SemaphoreType.DMA,
            pltpu.SemaphoreType.DMA,
            pltpu.SemaphoreType.DMA,
        ],
    )
    z32 = scatter(h32, zdest)

    zb = jax.lax.bitcast_convert_type(z32, jnp.bfloat16).reshape(T, E * F)
    wd_all = w_down.astype(jnp.bfloat16).reshape(E * F, D)

    out = pl.pallas_call(
        _down_body,
        grid=(T // _TMD, E),
        in_specs=[
            pl.BlockSpec((_TMD, F), lambda i, e: (i, e)),
            pl.BlockSpec((F, D), lambda i, e: (e, 0)),
        ],
        out_specs=pl.BlockSpec((_TMD, D), lambda i, e: (i, 0)),
        out_shape=jax.ShapeDtypeStruct((T, D), jnp.float32),
    )(zb, wd_all)
    return out


# speedup vs baseline: 31.7321x; 31.7321x over previous
import jax
import jax.numpy as jnp
from jax.experimental import pallas as pl
from jax.experimental.pallas import tpu as pltpu

T = 2048
D = 2048
E = 16
K = 8
F = 768

_RT = 256
_TM = 1024


def _router_body(x_ref, wr_ref, comb_ref):
    x = x_ref[...].astype(jnp.bfloat16)
    w = wr_ref[...].astype(jnp.bfloat16)
    logits = jax.lax.dot_general(
        x, w, (((1,), (0,)), ((), ())), preferred_element_type=jnp.float32)
    m = jnp.max(logits, axis=-1, keepdims=True)
    ex = jnp.exp(logits - m)
    probs = ex / jnp.sum(ex, axis=-1, keepdims=True)
    p = probs
    sel = jnp.zeros(p.shape, dtype=jnp.bool_)
    idx = jax.lax.broadcasted_iota(jnp.int32, p.shape, 1)
    for _ in range(K):
        mx = jnp.max(p, axis=-1, keepdims=True)
        ismx = p == mx
        first_idx = jnp.min(jnp.where(ismx, idx, E), axis=-1, keepdims=True)
        pick = idx == first_idx
        sel = sel | pick
        p = jnp.where(pick, -jnp.inf, p)
    w8 = jnp.where(sel, probs, 0.0)
    comb_ref[...] = w8 / jnp.sum(w8, axis=-1, keepdims=True)


def _ffn_body(xb_ref, wg_ref, wu_ref, wd_ref, comb_ref, out_ref):
    e = pl.program_id(1)
    x = xb_ref[...]
    g = jax.lax.dot_general(
        x, wg_ref[0], (((1,), (0,)), ((), ())), preferred_element_type=jnp.float32)
    u = jax.lax.dot_general(
        x, wu_ref[0], (((1,), (0,)), ((), ())), preferred_element_type=jnp.float32)
    h = (g * jax.lax.logistic(g)) * u
    y = jax.lax.dot_general(
        h.astype(jnp.bfloat16), wd_ref[0], (((1,), (0,)), ((), ())),
        preferred_element_type=jnp.float32)
    lane = jax.lax.broadcasted_iota(jnp.int32, (1, E), 1)
    c = jnp.sum(jnp.where(lane == e, comb_ref[...], 0.0), axis=1, keepdims=True)
    contrib = y * c

    @pl.when(e == 0)
    def _():
        out_ref[...] = contrib

    @pl.when(e != 0)
    def _():
        out_ref[...] += contrib


def kernel(x, W_router, w_gate, w_up, w_down):
    combine = pl.pallas_call(
        _router_body,
        grid=(T // _RT,),
        in_specs=[
            pl.BlockSpec((_RT, D), lambda i: (i, 0)),
            pl.BlockSpec((D, E), lambda i: (0, 0)),
        ],
        out_specs=pl.BlockSpec((_RT, E), lambda i: (i, 0)),
        out_shape=jax.ShapeDtypeStruct((T, E), jnp.float32),
    )(x, W_router)

    xb = x.astype(jnp.bfloat16)
    wgb = w_gate.astype(jnp.bfloat16)
    wub = w_up.astype(jnp.bfloat16)
    wdb = w_down.astype(jnp.bfloat16)

    out = pl.pallas_call(
        _ffn_body,
        grid=(T // _TM, E),
        in_specs=[
            pl.BlockSpec((_TM, D), lambda i, e: (i, 0)),
            pl.BlockSpec((1, D, F), lambda i, e: (e, 0, 0)),
            pl.BlockSpec((1, D, F), lambda i, e: (e, 0, 0)),
            pl.BlockSpec((1, F, D), lambda i, e: (e, 0, 0)),
            pl.BlockSpec((_TM, E), lambda i, e: (i, 0)),
        ],
        out_specs=pl.BlockSpec((_TM, D), lambda i, e: (i, 0)),
        out_shape=jax.ShapeDtypeStruct((T, D), jnp.float32),
    )(xb, wgb, wub, wdb, combine)
    return out


# speedup vs baseline: 31.7667x; 1.0011x over previous
import jax
import jax.numpy as jnp
from jax.experimental import pallas as pl

T = 2048
D = 2048
E = 16
K = 8
F = 768

_RT = 256
_TM = 1024


def _router_body(x_ref, wr_ref, comb_ref):
    x = x_ref[...].astype(jnp.bfloat16)
    w = wr_ref[...].astype(jnp.bfloat16)
    logits = jax.lax.dot_general(
        x, w, (((1,), (0,)), ((), ())), preferred_element_type=jnp.float32)
    m = jnp.max(logits, axis=-1, keepdims=True)
    ex = jnp.exp(logits - m)
    probs = ex / jnp.sum(ex, axis=-1, keepdims=True)
    p = probs
    sel = jnp.zeros(p.shape, dtype=jnp.bool_)
    idx = jax.lax.broadcasted_iota(jnp.int32, p.shape, 1)
    for _ in range(K):
        mx = jnp.max(p, axis=-1, keepdims=True)
        ismx = p == mx
        first_idx = jnp.min(jnp.where(ismx, idx, E), axis=-1, keepdims=True)
        pick = idx == first_idx
        sel = sel | pick
        p = jnp.where(pick, -jnp.inf, p)
    w8 = jnp.where(sel, probs, 0.0)
    comb_ref[...] = w8 / jnp.sum(w8, axis=-1, keepdims=True)


def _ffn_body(xb_ref, wg_ref, wu_ref, wd_ref, comb_ref, out_ref):
    e = pl.program_id(1)
    x = xb_ref[...]
    g = jax.lax.dot_general(
        x, wg_ref[0], (((1,), (0,)), ((), ())), preferred_element_type=jnp.float32)
    u = jax.lax.dot_general(
        x, wu_ref[0], (((1,), (0,)), ((), ())), preferred_element_type=jnp.float32)
    h = (g * jax.lax.logistic(g)) * u
    y = jax.lax.dot_general(
        h.astype(jnp.bfloat16), wd_ref[0], (((1,), (0,)), ((), ())),
        preferred_element_type=jnp.float32)
    lane = jax.lax.broadcasted_iota(jnp.int32, (1, E), 1)
    c = jnp.sum(jnp.where(lane == e, comb_ref[...], 0.0), axis=1, keepdims=True)
    contrib = y * c

    @pl.when(e == 0)
    def _():
        out_ref[...] = contrib

    @pl.when(e != 0)
    def _():
        out_ref[...] += contrib


def kernel(x, W_router, w_gate, w_up, w_down):
    combine = pl.pallas_call(
        _router_body,
        grid=(T // _RT,),
        in_specs=[
            pl.BlockSpec((_RT, D), lambda i: (i, 0)),
            pl.BlockSpec((D, E), lambda i: (0, 0)),
        ],
        out_specs=pl.BlockSpec((_RT, E), lambda i: (i, 0)),
        out_shape=jax.ShapeDtypeStruct((T, E), jnp.float32),
    )(x, W_router)

    xb = x.astype(jnp.bfloat16)
    wgb = w_gate.astype(jnp.bfloat16)
    wub = w_up.astype(jnp.bfloat16)
    wdb = w_down.astype(jnp.bfloat16)

    out = pl.pallas_call(
        _ffn_body,
        grid=(T // _TM, E),
        in_specs=[
            pl.BlockSpec((_TM, D), lambda i, e: (i, 0)),
            pl.BlockSpec((1, D, F), lambda i, e: (e, 0, 0)),
            pl.BlockSpec((1, D, F), lambda i, e: (e, 0, 0)),
            pl.BlockSpec((1, F, D), lambda i, e: (e, 0, 0)),
            pl.BlockSpec((_TM, E), lambda i, e: (i, 0)),
        ],
        out_specs=pl.BlockSpec((_TM, D), lambda i, e: (i, 0)),
        out_shape=jax.ShapeDtypeStruct((T, D), jnp.float32),
    )(xb, wgb, wub, wdb, combine)
    return out
